# baseline (speedup 1.0000x reference)
import functools

import jax
import jax.numpy as jnp
from jax import lax
from jax.experimental import pallas as pl
from jax.experimental.pallas import tpu as pltpu
from jax.experimental.pallas import tpu_sc as plsc

EMBED = 64
NUM_WORKERS = 32
CHUNK = 1600


def _gather_call(n_total, idx_flat, table):
    b_per_w = n_total // NUM_WORKERS
    n_chunks = b_per_w // CHUNK
    mesh = plsc.VectorSubcoreMesh(core_axis_name="c", subcore_axis_name="s")

    @functools.partial(
        pl.kernel,
        mesh=mesh,
        out_type=jax.ShapeDtypeStruct((n_total, EMBED), jnp.float32),
        scratch_types=[
            pltpu.VMEM((CHUNK,), jnp.int32),
            pltpu.VMEM((CHUNK, EMBED), jnp.float32),
            pltpu.SemaphoreType.DMA,
        ],
        compiler_params=pltpu.CompilerParams(use_tc_tiling_on_sc=False),
    )
    def grab(idx_hbm, tab_hbm, out_hbm, idx_v, rows_v, sem):
        wid = lax.axis_index("s") * 2 + lax.axis_index("c")
        base = wid * b_per_w

        def body(i, carry):
            off = base + i * CHUNK
            pltpu.sync_copy(idx_hbm.at[pl.ds(off, CHUNK)], idx_v)
            pltpu.async_copy(tab_hbm.at[idx_v], rows_v, sem).wait()
            pltpu.sync_copy(rows_v, out_hbm.at[pl.ds(off, CHUNK)])
            return carry

        lax.fori_loop(0, n_chunks, body, 0)

    return grab(idx_flat, table)


def kernel(input, table):
    b, h = input.shape
    n_total = b * h
    idx_flat = input.reshape(n_total).astype(jnp.int32)
    out = _gather_call(n_total, idx_flat, table)
    return out.reshape(b, h, EMBED)


# speedup vs baseline: 1.0075x; 1.0075x over previous
import functools

import jax
import jax.numpy as jnp
from jax import lax
from jax.experimental import pallas as pl
from jax.experimental.pallas import tpu as pltpu
from jax.experimental.pallas import tpu_sc as plsc

EMBED = 64
NUM_WORKERS = 32
CHUNK = 800
NBUF = 2


def _gather_call(n_total, idx_flat, table):
    b_per_w = n_total // NUM_WORKERS
    n_chunks = b_per_w // CHUNK
    n_groups = n_chunks // NBUF
    mesh = plsc.VectorSubcoreMesh(core_axis_name="c", subcore_axis_name="s")

    @functools.partial(
        pl.kernel,
        mesh=mesh,
        out_type=jax.ShapeDtypeStruct((n_total, EMBED), jnp.float32),
        scratch_types=[
            pltpu.VMEM((b_per_w,), jnp.int32),
            pltpu.VMEM((NBUF, CHUNK, EMBED), jnp.float32),
            pltpu.SemaphoreType.DMA,
            pltpu.SemaphoreType.DMA,
            pltpu.SemaphoreType.DMA,
            pltpu.SemaphoreType.DMA,
        ],
        compiler_params=pltpu.CompilerParams(use_tc_tiling_on_sc=False),
    )
    def grab(idx_hbm, tab_hbm, out_hbm, idx_v, rows_v, g0, g1, s0, s1):
        gsem = (g0, g1)
        ssem = (s0, s1)
        wid = lax.axis_index("s") * 2 + lax.axis_index("c")
        base = wid * b_per_w

        pltpu.sync_copy(idx_hbm.at[pl.ds(base, b_per_w)], idx_v)

        def start_gather(i, b):
            pltpu.async_copy(
                tab_hbm.at[idx_v.at[pl.ds(i * CHUNK, CHUNK)]],
                rows_v.at[b], gsem[b])

        def wait_gather(b):
            pltpu.make_async_copy(
                tab_hbm.at[idx_v.at[pl.ds(0, CHUNK)]], rows_v.at[b],
                gsem[b]).wait()

        def start_store(i, b):
            pltpu.async_copy(
                rows_v.at[b], out_hbm.at[pl.ds(base + i * CHUNK, CHUNK)],
                ssem[b])

        def wait_store(b):
            pltpu.make_async_copy(
                rows_v.at[b], out_hbm.at[pl.ds(base, CHUNK)], ssem[b]).wait()

        for b in range(NBUF):
            start_gather(b, b)

        def body(j, carry):
            i0 = j * NBUF
            for b in range(NBUF):
                i = i0 + b
                wait_gather(b)
                start_store(i, b)
                @pl.when(i + NBUF < n_chunks)
                def _():
                    wait_store(b)
                    start_gather(i + NBUF, b)
            return carry

        lax.fori_loop(0, n_groups, body, 0)
        for b in range(NBUF):
            wait_store(b)

    return grab(idx_flat, table)


def kernel(input, table):
    b, h = input.shape
    n_total = b * h
    idx_flat = input.reshape(n_total).astype(jnp.int32)
    out = _gather_call(n_total, idx_flat, table)
    return out.reshape(b, h, EMBED)


# speedup vs baseline: 1.0076x; 1.0001x over previous
import functools

import jax
import jax.numpy as jnp
from jax import lax
from jax.experimental import pallas as pl
from jax.experimental.pallas import tpu as pltpu
from jax.experimental.pallas import tpu_sc as plsc

EMBED = 64
NUM_WORKERS = 32
CHUNK = 400
NBUF = 4


def _gather_call(n_total, idx_flat, table):
    b_per_w = n_total // NUM_WORKERS
    n_chunks = b_per_w // CHUNK
    n_groups = n_chunks // NBUF
    mesh = plsc.VectorSubcoreMesh(core_axis_name="c", subcore_axis_name="s")

    @functools.partial(
        pl.kernel,
        mesh=mesh,
        out_type=jax.ShapeDtypeStruct((n_total, EMBED), jnp.float32),
        scratch_types=[
            pltpu.VMEM((b_per_w,), jnp.int32),
            pltpu.VMEM((NBUF, CHUNK, EMBED), jnp.float32),
        ] + [pltpu.SemaphoreType.DMA] * (2 * NBUF),
        compiler_params=pltpu.CompilerParams(use_tc_tiling_on_sc=False),
    )
    def grab(idx_hbm, tab_hbm, out_hbm, idx_v, rows_v, *sems):
        gsem = sems[:NBUF]
        ssem = sems[NBUF:]
        wid = lax.axis_index("s") * 2 + lax.axis_index("c")
        base = wid * b_per_w

        pltpu.sync_copy(idx_hbm.at[pl.ds(base, b_per_w)], idx_v)

        def start_gather(i, b):
            pltpu.async_copy(
                tab_hbm.at[idx_v.at[pl.ds(i * CHUNK, CHUNK)]],
                rows_v.at[b], gsem[b])

        def wait_gather(b):
            pltpu.make_async_copy(
                tab_hbm.at[idx_v.at[pl.ds(0, CHUNK)]], rows_v.at[b],
                gsem[b]).wait()

        def start_store(i, b):
            pltpu.async_copy(
                rows_v.at[b], out_hbm.at[pl.ds(base + i * CHUNK, CHUNK)],
                ssem[b])

        def wait_store(b):
            pltpu.make_async_copy(
                rows_v.at[b], out_hbm.at[pl.ds(base, CHUNK)], ssem[b]).wait()

        for b in range(NBUF):
            start_gather(b, b)

        def body(j, carry):
            i0 = j * NBUF
            for b in range(NBUF):
                i = i0 + b
                wait_gather(b)
                start_store(i, b)
                @pl.when(i + NBUF < n_chunks)
                def _():
                    wait_store(b)
                    start_gather(i + NBUF, b)
            return carry

        lax.fori_loop(0, n_groups, body, 0)
        for b in range(NBUF):
            wait_store(b)

    return grab(idx_flat, table)


def kernel(input, table):
    b, h = input.shape
    n_total = b * h
    idx_flat = input.reshape(n_total).astype(jnp.int32)
    out = _gather_call(n_total, idx_flat, table)
    return out.reshape(b, h, EMBED)


# speedup vs baseline: 1.0088x; 1.0013x over previous
import functools

import jax
import jax.numpy as jnp
from jax import lax
from jax.experimental import pallas as pl
from jax.experimental.pallas import tpu as pltpu
from jax.experimental.pallas import tpu_sc as plsc

EMBED = 64
NUM_WORKERS = 32
CHUNK = 400
NBUF = 4


def _gather_call(n_batch, n_hist, idx_flat, table):
    n_total = n_batch * n_hist
    b_per_w = n_total // NUM_WORKERS
    n_chunks = b_per_w // CHUNK
    n_groups = n_chunks // NBUF
    mesh = plsc.VectorSubcoreMesh(core_axis_name="c", subcore_axis_name="s")

    @functools.partial(
        pl.kernel,
        mesh=mesh,
        out_type=jax.ShapeDtypeStruct((n_batch, n_hist, EMBED), jnp.float32),
        scratch_types=[
            pltpu.VMEM((b_per_w,), jnp.int32),
            pltpu.VMEM((NBUF, CHUNK, EMBED), jnp.float32),
        ] + [pltpu.SemaphoreType.DMA] * (2 * NBUF),
        compiler_params=pltpu.CompilerParams(use_tc_tiling_on_sc=False),
    )
    def grab(idx_hbm, tab_hbm, out_hbm, idx_v, rows_v, *sems):
        gsem = sems[:NBUF]
        ssem = sems[NBUF:]
        wid = lax.axis_index("s") * 2 + lax.axis_index("c")
        base = wid * b_per_w

        pltpu.sync_copy(idx_hbm.at[pl.ds(base, b_per_w)], idx_v)

        nb = CHUNK // 200
        b0 = wid * (b_per_w // 200)

        def start_gather(i, b):
            pltpu.async_copy(
                tab_hbm.at[idx_v.at[pl.ds(i * CHUNK, CHUNK)]],
                rows_v.at[b], gsem[b])

        def wait_gather(b):
            pltpu.make_async_copy(
                tab_hbm.at[idx_v.at[pl.ds(0, CHUNK)]], rows_v.at[b],
                gsem[b]).wait()

        def start_store(i, b):
            for k in range(nb):
                pltpu.async_copy(
                    rows_v.at[b, pl.ds(k * n_hist, n_hist)],
                    out_hbm.at[b0 + i * nb + k], ssem[b])

        def wait_store(b):
            pltpu.make_async_copy(
                tab_hbm.at[pl.ds(0, CHUNK)], rows_v.at[b], ssem[b]).wait()

        for b in range(NBUF):
            start_gather(b, b)

        def body(j, carry):
            i0 = j * NBUF
            for b in range(NBUF):
                i = i0 + b
                wait_gather(b)
                start_store(i, b)
                @pl.when(i + NBUF < n_chunks)
                def _():
                    wait_store(b)
                    start_gather(i + NBUF, b)
            return carry

        lax.fori_loop(0, n_groups, body, 0)
        for b in range(NBUF):
            wait_store(b)

    return grab(idx_flat, table)


def kernel(input, table):
    b, h = input.shape
    idx_flat = input.reshape(b * h).astype(jnp.int32)
    return _gather_call(b, h, idx_flat, table)
